# baseline (speedup 1.0000x reference)
import functools

import jax
import jax.numpy as jnp
from jax import lax
from jax.experimental import pallas as pl
from jax.experimental.pallas import tpu as pltpu
from jax.experimental.pallas import tpu_sc as plsc

ATOM_DIM = 32
BOND_DIM = 16
N_NODES = 10000
N_EDGES = 100000

NW = 32
CHUNK = 128
E_PER_W = 3200
E_PAD = NW * E_PER_W
N_CHUNKS = E_PER_W // CHUNK
N_PAD = 10240
ROWS_PER_TILE = N_PAD // 16


def _gather_body(table_hbm, idx_hbm, out_hbm, idx_v, rows_v, sem):
    cid = lax.axis_index("c")
    sid = lax.axis_index("s")
    wid = sid * 2 + cid
    base = wid * E_PER_W
    pltpu.sync_copy(idx_hbm.at[pl.ds(wid * N_CHUNKS, N_CHUNKS)], idx_v)
    copies = [
        pltpu.async_copy(table_hbm.at[idx_v.at[j]],
                         rows_v.at[pl.ds(j * CHUNK, CHUNK)], sem)
        for j in range(N_CHUNKS)
    ]
    for c in copies:
        c.wait()
    pltpu.sync_copy(rows_v, out_hbm.at[pl.ds(base, E_PER_W)])


def _sc_gather(atom_features, src_idx2d):
    k = functools.partial(
        pl.kernel,
        out_type=jax.ShapeDtypeStruct((E_PAD, ATOM_DIM), jnp.float32),
        mesh=plsc.VectorSubcoreMesh(core_axis_name="c", subcore_axis_name="s"),
        scratch_types=[
            pltpu.VMEM((N_CHUNKS, CHUNK), jnp.int32),
            pltpu.VMEM((E_PER_W, ATOM_DIM), jnp.float32),
            pltpu.SemaphoreType.DMA,
        ],
        compiler_params=pltpu.CompilerParams(use_tc_tiling_on_sc=False),
    )(_gather_body)
    return k(atom_features, src_idx2d)


def _scatter_body(msg_hbm, idx_hbm, zeros_hbm, out_hbm,
                  msg_v, idx_small, acc_shared):
    cid = lax.axis_index("c")
    sid = lax.axis_index("s")
    wid = sid * 2 + cid
    base = wid * E_PER_W
    row0 = sid * ROWS_PER_TILE
    pltpu.sync_copy(msg_hbm.at[pl.ds(base, E_PER_W)], msg_v)
    pltpu.sync_copy(zeros_hbm.at[pl.ds(row0, ROWS_PER_TILE)],
                    acc_shared.at[pl.ds(row0, ROWS_PER_TILE)])
    plsc.subcore_barrier()

    def step(j, carry):
        off = j * CHUNK
        pltpu.sync_copy(idx_hbm.at[pl.ds(base + off, CHUNK)], idx_small)
        pltpu.sync_copy(msg_v.at[pl.ds(off, CHUNK)],
                        acc_shared.at[idx_small], add=True)
        return carry

    lax.fori_loop(0, N_CHUNKS, step, 0)
    plsc.subcore_barrier()
    pltpu.sync_copy(acc_shared.at[pl.ds(row0, ROWS_PER_TILE)],
                    out_hbm.at[pl.ds(cid * N_PAD + row0, ROWS_PER_TILE)])


def _sc_scatter(msg, dst_idx, zeros_init):
    k = functools.partial(
        pl.kernel,
        out_type=jax.ShapeDtypeStruct((2 * N_PAD, ATOM_DIM), jnp.float32),
        mesh=plsc.VectorSubcoreMesh(core_axis_name="c", subcore_axis_name="s"),
        scratch_types=[
            pltpu.VMEM((E_PER_W, ATOM_DIM), jnp.float32),
            pltpu.VMEM((CHUNK,), jnp.int32),
            pltpu.VMEM_SHARED((N_PAD, ATOM_DIM), jnp.float32),
        ],
        compiler_params=pltpu.CompilerParams(use_tc_tiling_on_sc=False),
    )(_scatter_body)
    return k(msg, dst_idx, zeros_init)


def _matmul_body(x_ref, bond_ref, r_ref, s_ref, w4_ref, bt_ref, out_ref):
    x = x_ref[...]
    o = (jnp.dot(bond_ref[...], r_ref[...],
                 preferred_element_type=jnp.float32)
         * jnp.dot(x, s_ref[...], preferred_element_type=jnp.float32))
    msg = (jnp.dot(o, w4_ref[...], preferred_element_type=jnp.float32)
           + jnp.dot(x, bt_ref[...], preferred_element_type=jnp.float32))
    out_ref[...] = msg


def _tc_matmul(xg, bond_pad, rsel, ssel, w4, bias_t):
    tile = 800
    grid = (E_PAD // tile,)
    return pl.pallas_call(
        _matmul_body,
        grid=grid,
        in_specs=[
            pl.BlockSpec((tile, ATOM_DIM), lambda i: (i, 0)),
            pl.BlockSpec((tile, BOND_DIM),
                         lambda i: (jnp.minimum(i, (N_EDGES // tile) - 1), 0)),
            pl.BlockSpec((BOND_DIM, 512), lambda i: (0, 0)),
            pl.BlockSpec((ATOM_DIM, 512), lambda i: (0, 0)),
            pl.BlockSpec((512, ATOM_DIM), lambda i: (0, 0)),
            pl.BlockSpec((ATOM_DIM, ATOM_DIM), lambda i: (0, 0)),
        ],
        out_specs=pl.BlockSpec((tile, ATOM_DIM), lambda i: (i, 0)),
        out_shape=jax.ShapeDtypeStruct((E_PAD, ATOM_DIM), jnp.float32),
    )(xg, bond_pad, rsel, ssel, w4, bias_t)


def _combine_body(p_ref, out_ref):
    out_ref[...] = p_ref[0] + p_ref[1]


def _tc_combine(partials):
    return pl.pallas_call(
        _combine_body,
        out_shape=jax.ShapeDtypeStruct((N_PAD, ATOM_DIM), jnp.float32),
    )(partials)


def kernel(atom_features, bond_features, pair_indices, kernel, bias):
    weight = kernel
    src = pair_indices[:, 1].astype(jnp.int32)
    dst = pair_indices[:, 0].astype(jnp.int32)
    src_pad = jnp.concatenate(
        [src, jnp.zeros((E_PAD - N_EDGES,), jnp.int32)])
    dst_pad = jnp.concatenate(
        [dst, jnp.full((E_PAD - N_EDGES,), N_NODES, jnp.int32)])
    rsel = jnp.kron(jnp.eye(BOND_DIM, dtype=jnp.float32),
                    jnp.ones((1, ATOM_DIM), jnp.float32))
    ssel = jnp.kron(jnp.ones((1, BOND_DIM), jnp.float32),
                    jnp.eye(ATOM_DIM, dtype=jnp.float32))
    w3 = weight.reshape(BOND_DIM, ATOM_DIM, ATOM_DIM)
    w4 = jnp.transpose(w3, (0, 2, 1)).reshape(BOND_DIM * ATOM_DIM, ATOM_DIM)
    bias_t = bias.reshape(ATOM_DIM, ATOM_DIM).T

    xg = _sc_gather(atom_features, src_pad.reshape(E_PAD // CHUNK, CHUNK))
    msg = _tc_matmul(xg, bond_features, rsel, ssel, w4, bias_t)
    zeros_init = jnp.zeros((N_PAD, ATOM_DIM), jnp.float32)
    partials = _sc_scatter(msg, dst_pad, zeros_init)
    out = _tc_combine(partials.reshape(2, N_PAD, ATOM_DIM))
    return out[:N_NODES]


# speedup vs baseline: 1.2632x; 1.2632x over previous
import functools

import jax
import jax.numpy as jnp
from jax import lax
from jax.experimental import pallas as pl
from jax.experimental.pallas import tpu as pltpu
from jax.experimental.pallas import tpu_sc as plsc

ATOM_DIM = 32
BOND_DIM = 16
N_NODES = 10000
N_EDGES = 100000

NW = 32
CHUNK = 128
E_PER_W = 3200
E_PAD = NW * E_PER_W
N_CHUNKS = E_PER_W // CHUNK
N_PAD = 10240
ROWS_PER_TILE = N_PAD // 16


def _gather_body(table_hbm, idx_hbm, out_hbm, idx_v, rows_v, sem):
    cid = lax.axis_index("c")
    sid = lax.axis_index("s")
    wid = sid * 2 + cid
    base = wid * E_PER_W
    pltpu.sync_copy(idx_hbm.at[pl.ds(wid * N_CHUNKS, N_CHUNKS)], idx_v)
    copies = [
        pltpu.async_copy(table_hbm.at[idx_v.at[j]],
                         rows_v.at[pl.ds(j * CHUNK, CHUNK)], sem)
        for j in range(N_CHUNKS)
    ]
    for c in copies:
        c.wait()
    pltpu.sync_copy(rows_v, out_hbm.at[pl.ds(base, E_PER_W)])


def _sc_gather(atom_features, src_idx2d):
    k = functools.partial(
        pl.kernel,
        out_type=jax.ShapeDtypeStruct((E_PAD, ATOM_DIM), jnp.float32),
        mesh=plsc.VectorSubcoreMesh(core_axis_name="c", subcore_axis_name="s"),
        scratch_types=[
            pltpu.VMEM((N_CHUNKS, CHUNK), jnp.int32),
            pltpu.VMEM((E_PER_W, ATOM_DIM), jnp.float32),
            pltpu.SemaphoreType.DMA,
        ],
        compiler_params=pltpu.CompilerParams(use_tc_tiling_on_sc=False),
    )(_gather_body)
    return k(atom_features, src_idx2d)


def _scatter_body(msg_hbm, idx_hbm, zeros_hbm, out_hbm,
                  msg_v, idx_small, acc_shared):
    cid = lax.axis_index("c")
    sid = lax.axis_index("s")
    wid = sid * 2 + cid
    base = wid * E_PER_W
    row0 = sid * ROWS_PER_TILE
    pltpu.sync_copy(msg_hbm.at[pl.ds(base, E_PER_W)], msg_v)
    pltpu.sync_copy(zeros_hbm.at[pl.ds(row0, ROWS_PER_TILE)],
                    acc_shared.at[pl.ds(row0, ROWS_PER_TILE)])
    plsc.subcore_barrier()

    def step(j, carry):
        off = j * CHUNK
        pltpu.sync_copy(idx_hbm.at[pl.ds(base + off, CHUNK)], idx_small)
        pltpu.sync_copy(msg_v.at[pl.ds(off, CHUNK)],
                        acc_shared.at[idx_small], add=True)
        return carry

    lax.fori_loop(0, N_CHUNKS, step, 0)
    plsc.subcore_barrier()
    pltpu.sync_copy(acc_shared.at[pl.ds(row0, ROWS_PER_TILE)],
                    out_hbm.at[pl.ds(cid * N_PAD + row0, ROWS_PER_TILE)])


def _sc_scatter(msg, dst_idx, zeros_init):
    k = functools.partial(
        pl.kernel,
        out_type=jax.ShapeDtypeStruct((2 * N_PAD, ATOM_DIM), jnp.float32),
        mesh=plsc.VectorSubcoreMesh(core_axis_name="c", subcore_axis_name="s"),
        scratch_types=[
            pltpu.VMEM((E_PER_W, ATOM_DIM), jnp.float32),
            pltpu.VMEM((CHUNK,), jnp.int32),
            pltpu.VMEM_SHARED((N_PAD, ATOM_DIM), jnp.float32),
        ],
        compiler_params=pltpu.CompilerParams(use_tc_tiling_on_sc=False),
    )(_scatter_body)
    return k(msg, dst_idx, zeros_init)


def _matmul_body(x_ref, bond_ref, r_ref, s_ref, w4_ref, bt_ref, out_ref):
    z = x_ref[...]
    x = jnp.concatenate([z[:, 32 * c:32 * c + 32] for c in range(4)],
                        axis=0)
    ob = lax.dot_general(bond_ref[...], r_ref[...],
                         (((0,), (0,)), ((), ())),
                         preferred_element_type=jnp.float32)
    o = ob * jnp.dot(x, s_ref[...], preferred_element_type=jnp.float32)
    msg = (jnp.dot(o, w4_ref[...], preferred_element_type=jnp.float32)
           + jnp.dot(x, bt_ref[...], preferred_element_type=jnp.float32))
    q = msg.shape[0] // 4
    out_ref[...] = jnp.concatenate(
        [msg[q * c:q * c + q, :] for c in range(4)], axis=1)


def _tc_matmul(xg_packed, bond_t, rsel, ssel, w4, bias_t):
    tile = 2048
    grid = (E_PAD // tile,)
    return pl.pallas_call(
        _matmul_body,
        grid=grid,
        in_specs=[
            pl.BlockSpec((tile // 4, 128), lambda i: (i, 0)),
            pl.BlockSpec((BOND_DIM, tile), lambda i: (0, i)),
            pl.BlockSpec((BOND_DIM, 512), lambda i: (0, 0)),
            pl.BlockSpec((ATOM_DIM, 512), lambda i: (0, 0)),
            pl.BlockSpec((512, ATOM_DIM), lambda i: (0, 0)),
            pl.BlockSpec((ATOM_DIM, ATOM_DIM), lambda i: (0, 0)),
        ],
        out_specs=pl.BlockSpec((tile // 4, 128), lambda i: (i, 0)),
        out_shape=jax.ShapeDtypeStruct((E_PAD // 4, 128), jnp.float32),
    )(xg_packed, bond_t, rsel, ssel, w4, bias_t)


def _combine_body(p_ref, out_ref):
    out_ref[...] = p_ref[0] + p_ref[1]


def _tc_combine(partials):
    return pl.pallas_call(
        _combine_body,
        out_shape=jax.ShapeDtypeStruct((N_PAD, ATOM_DIM), jnp.float32),
    )(partials)


def kernel(atom_features, bond_features, pair_indices, kernel, bias):
    weight = kernel
    src = pair_indices[:, 1].astype(jnp.int32)
    dst = pair_indices[:, 0].astype(jnp.int32)
    src_pad = jnp.concatenate(
        [src, jnp.zeros((E_PAD - N_EDGES,), jnp.int32)])
    dst_pad = jnp.concatenate(
        [dst, jnp.full((E_PAD - N_EDGES,), N_NODES, jnp.int32)])
    bond_t = jnp.pad(bond_features.T, ((0, 0), (0, E_PAD - N_EDGES)))
    tile = 2048
    bond_t = (bond_t.reshape(BOND_DIM, E_PAD // tile, tile // 4, 4)
              .transpose(0, 1, 3, 2).reshape(BOND_DIM, E_PAD))
    rsel = jnp.kron(jnp.eye(BOND_DIM, dtype=jnp.float32),
                    jnp.ones((1, ATOM_DIM), jnp.float32))
    ssel = jnp.kron(jnp.ones((1, BOND_DIM), jnp.float32),
                    jnp.eye(ATOM_DIM, dtype=jnp.float32))
    w3 = weight.reshape(BOND_DIM, ATOM_DIM, ATOM_DIM)
    w4 = jnp.transpose(w3, (0, 2, 1)).reshape(BOND_DIM * ATOM_DIM, ATOM_DIM)
    bias_t = bias.reshape(ATOM_DIM, ATOM_DIM).T

    xg = _sc_gather(atom_features, src_pad.reshape(E_PAD // CHUNK, CHUNK))
    msg = _tc_matmul(xg.reshape(E_PAD // 4, 128), bond_t,
                     rsel, ssel, w4, bias_t)
    msg = msg.reshape(E_PAD, ATOM_DIM)
    zeros_init = jnp.zeros((N_PAD, ATOM_DIM), jnp.float32)
    partials = _sc_scatter(msg, dst_pad, zeros_init)
    out = _tc_combine(partials.reshape(2, N_PAD, ATOM_DIM))
    return out[:N_NODES]


# speedup vs baseline: 1.2980x; 1.0275x over previous
import functools

import jax
import jax.numpy as jnp
from jax import lax
from jax.experimental import pallas as pl
from jax.experimental.pallas import tpu as pltpu
from jax.experimental.pallas import tpu_sc as plsc

ATOM_DIM = 32
BOND_DIM = 16
N_NODES = 10000
N_EDGES = 100000

NW = 32
CHUNK = 128
E_PER_W = 3200
E_PAD = NW * E_PER_W
N_CHUNKS = E_PER_W // CHUNK
N_PAD = 10240
ROWS_PER_TILE = N_PAD // 16


def _gather_body(table_hbm, idx_hbm, out_hbm, idx_v, rows_v, sem):
    cid = lax.axis_index("c")
    sid = lax.axis_index("s")
    wid = sid * 2 + cid
    base = wid * E_PER_W
    pltpu.sync_copy(idx_hbm.at[pl.ds(wid * N_CHUNKS, N_CHUNKS)], idx_v)
    copies = [
        pltpu.async_copy(table_hbm.at[idx_v.at[j]],
                         rows_v.at[pl.ds(j * CHUNK, CHUNK)], sem)
        for j in range(N_CHUNKS)
    ]
    for c in copies:
        c.wait()
    pltpu.sync_copy(rows_v, out_hbm.at[pl.ds(base, E_PER_W)])


def _sc_gather(atom_features, src_idx2d):
    k = functools.partial(
        pl.kernel,
        out_type=jax.ShapeDtypeStruct((E_PAD, ATOM_DIM), jnp.float32),
        mesh=plsc.VectorSubcoreMesh(core_axis_name="c", subcore_axis_name="s"),
        scratch_types=[
            pltpu.VMEM((N_CHUNKS, CHUNK), jnp.int32),
            pltpu.VMEM((E_PER_W, ATOM_DIM), jnp.float32),
            pltpu.SemaphoreType.DMA,
        ],
        compiler_params=pltpu.CompilerParams(use_tc_tiling_on_sc=False),
    )(_gather_body)
    return k(atom_features, src_idx2d)


def _scatter_body(msg_hbm, idx_hbm, zeros_hbm, out_hbm,
                  msg_v, idx_small, acc_shared):
    cid = lax.axis_index("c")
    sid = lax.axis_index("s")
    wid = sid * 2 + cid
    base = wid * E_PER_W
    row0 = sid * ROWS_PER_TILE
    pltpu.sync_copy(msg_hbm.at[pl.ds(base, E_PER_W)], msg_v)
    pltpu.sync_copy(zeros_hbm.at[pl.ds(row0, ROWS_PER_TILE)],
                    acc_shared.at[pl.ds(row0, ROWS_PER_TILE)])
    plsc.subcore_barrier()

    def step(j, carry):
        off = j * CHUNK
        pltpu.sync_copy(idx_hbm.at[pl.ds(base + off, CHUNK)], idx_small)
        pltpu.sync_copy(msg_v.at[pl.ds(off, CHUNK)],
                        acc_shared.at[idx_small], add=True)
        return carry

    lax.fori_loop(0, N_CHUNKS, step, 0)
    plsc.subcore_barrier()
    pltpu.sync_copy(acc_shared.at[pl.ds(row0, ROWS_PER_TILE)],
                    out_hbm.at[pl.ds(cid * N_PAD + row0, ROWS_PER_TILE)])


def _sc_scatter(msg, dst_idx, zeros_init):
    k = functools.partial(
        pl.kernel,
        out_type=jax.ShapeDtypeStruct((2 * N_PAD, ATOM_DIM), jnp.float32),
        mesh=plsc.VectorSubcoreMesh(core_axis_name="c", subcore_axis_name="s"),
        scratch_types=[
            pltpu.VMEM((E_PER_W, ATOM_DIM), jnp.float32),
            pltpu.VMEM((CHUNK,), jnp.int32),
            pltpu.VMEM_SHARED((N_PAD, ATOM_DIM), jnp.float32),
        ],
        compiler_params=pltpu.CompilerParams(use_tc_tiling_on_sc=False),
    )(_scatter_body)
    return k(msg, dst_idx, zeros_init)


def _matmul_body(x_ref, bm_ref, bt_ref2, r_ref, s_ref, w4_ref, bt_ref,
                 out_ref):
    z = x_ref[...]
    x = jnp.concatenate([z[:, 32 * c:32 * c + 32] for c in range(4)],
                        axis=0)
    i = pl.program_id(0)
    nfull = N_EDGES // (z.shape[0] * 4)
    b = jnp.where(i < nfull, bm_ref[...], bt_ref2[...])
    ob = jnp.dot(b, r_ref[...], preferred_element_type=jnp.float32)
    o = ob * jnp.dot(x, s_ref[...], preferred_element_type=jnp.float32)
    msg = (jnp.dot(o, w4_ref[...], preferred_element_type=jnp.float32)
           + jnp.dot(x, bt_ref[...], preferred_element_type=jnp.float32))
    q = msg.shape[0] // 4
    out_ref[...] = jnp.concatenate(
        [msg[q * c:q * c + q, :] for c in range(4)], axis=1)


def _tc_matmul(xg_packed, bond, bond_tail, rsel, ssel, w4, bias_t):
    tile = 2048
    grid = (E_PAD // tile,)
    nfull = N_EDGES // tile
    return pl.pallas_call(
        _matmul_body,
        grid=grid,
        in_specs=[
            pl.BlockSpec((tile // 4, 128), lambda i: (i, 0)),
            pl.BlockSpec((tile, BOND_DIM),
                         lambda i: (jnp.minimum(i, nfull - 1), 0)),
            pl.BlockSpec((tile, BOND_DIM),
                         lambda i: (jnp.maximum(i - nfull, 0), 0)),
            pl.BlockSpec((BOND_DIM, 512), lambda i: (0, 0)),
            pl.BlockSpec((ATOM_DIM, 512), lambda i: (0, 0)),
            pl.BlockSpec((512, ATOM_DIM), lambda i: (0, 0)),
            pl.BlockSpec((ATOM_DIM, ATOM_DIM), lambda i: (0, 0)),
        ],
        out_specs=pl.BlockSpec((tile // 4, 128), lambda i: (i, 0)),
        out_shape=jax.ShapeDtypeStruct((E_PAD // 4, 128), jnp.float32),
    )(xg_packed, bond, bond_tail, rsel, ssel, w4, bias_t)


def _combine_body(p_ref, out_ref):
    out_ref[...] = p_ref[0] + p_ref[1]


def _tc_combine(partials):
    return pl.pallas_call(
        _combine_body,
        out_shape=jax.ShapeDtypeStruct((N_PAD, ATOM_DIM), jnp.float32),
    )(partials)


def kernel(atom_features, bond_features, pair_indices, kernel, bias):
    weight = kernel
    src = pair_indices[:, 1].astype(jnp.int32)
    dst = pair_indices[:, 0].astype(jnp.int32)
    src_pad = jnp.concatenate(
        [src, jnp.zeros((E_PAD - N_EDGES,), jnp.int32)])
    dst_pad = jnp.concatenate(
        [dst, jnp.full((E_PAD - N_EDGES,), N_NODES, jnp.int32)])
    tile = 2048
    def _perm(a):
        return (a.reshape(E_PAD // tile, 4, tile // 4)
                .transpose(0, 2, 1).reshape(E_PAD))
    src_perm = _perm(src_pad)
    dst_perm = _perm(dst_pad)
    bond_tail = jnp.pad(bond_features[tile * (N_EDGES // tile):],
                        ((0, 2 * tile - (N_EDGES % tile)), (0, 0)))
    rsel = jnp.kron(jnp.eye(BOND_DIM, dtype=jnp.float32),
                    jnp.ones((1, ATOM_DIM), jnp.float32))
    ssel = jnp.kron(jnp.ones((1, BOND_DIM), jnp.float32),
                    jnp.eye(ATOM_DIM, dtype=jnp.float32))
    w3 = weight.reshape(BOND_DIM, ATOM_DIM, ATOM_DIM)
    w4 = jnp.transpose(w3, (0, 2, 1)).reshape(BOND_DIM * ATOM_DIM, ATOM_DIM)
    bias_t = bias.reshape(ATOM_DIM, ATOM_DIM).T

    xg = _sc_gather(atom_features, src_perm.reshape(E_PAD // CHUNK, CHUNK))
    msg = _tc_matmul(xg.reshape(E_PAD // 4, 128), bond_features, bond_tail,
                     rsel, ssel, w4, bias_t)
    msg = msg.reshape(E_PAD, ATOM_DIM)
    zeros_init = jnp.zeros((N_PAD, ATOM_DIM), jnp.float32)
    partials = _sc_scatter(msg, dst_perm, zeros_init)
    out = _tc_combine(partials.reshape(2, N_PAD, ATOM_DIM))
    return out[:N_NODES]


# speedup vs baseline: 1.6301x; 1.2558x over previous
import functools

import jax
import jax.numpy as jnp
from jax import lax
from jax.experimental import pallas as pl
from jax.experimental.pallas import tpu as pltpu
from jax.experimental.pallas import tpu_sc as plsc

ATOM_DIM = 32
BOND_DIM = 16
N_NODES = 10000
N_EDGES = 100000

NW = 32
CHUNK = 128
E_PER_W = 3200
E_PAD = NW * E_PER_W
N_CHUNKS = E_PER_W // CHUNK
N_PAD = 10240
ROWS_PER_TILE = N_PAD // 16


def _permute_idx(idx_lin, idx_p):
    q = E_PER_W // 4

    def body(m, carry):
        k = m * 16 + lax.iota(jnp.int32, 16)
        off = (k >> 2) + (k & 3) * q
        v = plsc.load_gather(idx_lin, [off])
        idx_p[m // 8, pl.ds((m % 8) * 16, 16)] = v
        return carry

    lax.fori_loop(0, E_PER_W // 16, body, 0)


def _gather_body(table_hbm, idx_hbm, out_hbm, idx_lin, idx_p, rows_v, sem):
    cid = lax.axis_index("c")
    sid = lax.axis_index("s")
    wid = sid * 2 + cid
    base = wid * E_PER_W
    pltpu.sync_copy(idx_hbm.at[pl.ds(base, E_PER_W)], idx_lin)
    _permute_idx(idx_lin, idx_p)
    copies = [
        pltpu.async_copy(table_hbm.at[idx_p.at[j]],
                         rows_v.at[pl.ds(j * CHUNK, CHUNK)], sem)
        for j in range(N_CHUNKS)
    ]
    for c in copies:
        c.wait()
    pltpu.sync_copy(rows_v, out_hbm.at[pl.ds(base, E_PER_W)])


def _sc_gather(atom_features, src_idx):
    k = functools.partial(
        pl.kernel,
        out_type=jax.ShapeDtypeStruct((E_PAD, ATOM_DIM), jnp.float32),
        mesh=plsc.VectorSubcoreMesh(core_axis_name="c", subcore_axis_name="s"),
        scratch_types=[
            pltpu.VMEM((E_PER_W,), jnp.int32),
            pltpu.VMEM((N_CHUNKS, CHUNK), jnp.int32),
            pltpu.VMEM((E_PER_W, ATOM_DIM), jnp.float32),
            pltpu.SemaphoreType.DMA,
        ],
        compiler_params=pltpu.CompilerParams(use_tc_tiling_on_sc=False, needs_layout_passes=False),
    )(_gather_body)
    return k(atom_features, src_idx)


def _scatter_body(msg_hbm, idx_hbm, zeros_hbm, out_hbm,
                  msg_v, idx_lin, idx_p, acc_shared):
    cid = lax.axis_index("c")
    sid = lax.axis_index("s")
    wid = sid * 2 + cid
    base = wid * E_PER_W
    row0 = sid * ROWS_PER_TILE
    pltpu.sync_copy(idx_hbm.at[pl.ds(base, E_PER_W)], idx_lin)
    pltpu.sync_copy(msg_hbm.at[pl.ds(base, E_PER_W)], msg_v)
    _permute_idx(idx_lin, idx_p)
    pltpu.sync_copy(zeros_hbm.at[pl.ds(row0, ROWS_PER_TILE)],
                    acc_shared.at[pl.ds(row0, ROWS_PER_TILE)])
    plsc.subcore_barrier()

    def step(j, carry):
        off = j * CHUNK
        pltpu.sync_copy(msg_v.at[pl.ds(off, CHUNK)],
                        acc_shared.at[idx_p.at[j]], add=True)
        return carry

    lax.fori_loop(0, N_CHUNKS, step, 0)
    plsc.subcore_barrier()
    pltpu.sync_copy(acc_shared.at[pl.ds(row0, ROWS_PER_TILE)],
                    out_hbm.at[pl.ds(cid * N_PAD + row0, ROWS_PER_TILE)])


def _sc_scatter(msg, dst_idx, zeros_init):
    k = functools.partial(
        pl.kernel,
        out_type=jax.ShapeDtypeStruct((2 * N_PAD, ATOM_DIM), jnp.float32),
        mesh=plsc.VectorSubcoreMesh(core_axis_name="c", subcore_axis_name="s"),
        scratch_types=[
            pltpu.VMEM((E_PER_W, ATOM_DIM), jnp.float32),
            pltpu.VMEM((E_PER_W,), jnp.int32),
            pltpu.VMEM((N_CHUNKS, CHUNK), jnp.int32),
            pltpu.VMEM_SHARED((N_PAD, ATOM_DIM), jnp.float32),
        ],
        compiler_params=pltpu.CompilerParams(use_tc_tiling_on_sc=False, needs_layout_passes=False),
    )(_scatter_body)
    return k(msg, dst_idx, zeros_init)


def _matmul_body(x_ref, bm_ref, btl_ref, r_ref, s_ref, w4_ref, bt_ref,
                 out_ref):
    z = x_ref[...]
    x = jnp.concatenate([z[:, 32 * c:32 * c + 32] for c in range(4)],
                        axis=0)
    i = pl.program_id(0)
    tile = z.shape[0] * 4
    nfull = N_EDGES // tile
    b = jnp.where(i < nfull, bm_ref[...], btl_ref[...])
    ob = lax.dot_general(b, r_ref[...], (((0,), (0,)), ((), ())),
                         preferred_element_type=jnp.float32)
    o = ob * jnp.dot(x, s_ref[...], preferred_element_type=jnp.float32)
    msg = (jnp.dot(o, w4_ref[...], preferred_element_type=jnp.float32)
           + jnp.dot(x, bt_ref[...], preferred_element_type=jnp.float32))
    q = msg.shape[0] // 4
    out_ref[...] = jnp.concatenate(
        [msg[q * c:q * c + q, :] for c in range(4)], axis=1)


def _tc_matmul(xg_packed, bond_t, bond_tail_t, rsel, ssel, w4, bias_t):
    tile = E_PER_W
    grid = (E_PAD // tile,)
    nfull = N_EDGES // tile
    return pl.pallas_call(
        _matmul_body,
        grid=grid,
        in_specs=[
            pl.BlockSpec((tile // 4, 128), lambda i: (i, 0)),
            pl.BlockSpec((BOND_DIM, tile),
                         lambda i: (0, jnp.minimum(i, nfull - 1))),
            pl.BlockSpec((BOND_DIM, tile), lambda i: (0, 0)),
            pl.BlockSpec((BOND_DIM, 512), lambda i: (0, 0)),
            pl.BlockSpec((ATOM_DIM, 512), lambda i: (0, 0)),
            pl.BlockSpec((512, ATOM_DIM), lambda i: (0, 0)),
            pl.BlockSpec((ATOM_DIM, ATOM_DIM), lambda i: (0, 0)),
        ],
        out_specs=pl.BlockSpec((tile // 4, 128), lambda i: (i, 0)),
        out_shape=jax.ShapeDtypeStruct((E_PAD // 4, 128), jnp.float32),
    )(xg_packed, bond_t, bond_tail_t, rsel, ssel, w4, bias_t)


def _combine_body(p_ref, out_ref):
    out_ref[...] = p_ref[0] + p_ref[1]


def _tc_combine(partials):
    return pl.pallas_call(
        _combine_body,
        out_shape=jax.ShapeDtypeStruct((N_PAD, ATOM_DIM), jnp.float32),
    )(partials)


def kernel(atom_features, bond_features, pair_indices, kernel, bias):
    weight = kernel
    src = pair_indices[:, 1].astype(jnp.int32)
    dst = pair_indices[:, 0].astype(jnp.int32)
    src_pad = jnp.concatenate(
        [src, jnp.zeros((E_PAD - N_EDGES,), jnp.int32)])
    dst_pad = jnp.concatenate(
        [dst, jnp.full((E_PAD - N_EDGES,), N_NODES, jnp.int32)])
    tile = E_PER_W
    nfull = N_EDGES // tile
    bond_t = bond_features.T
    bond_tail_t = jnp.pad(bond_t[:, nfull * tile:],
                          ((0, 0), (0, (nfull + 1) * tile - N_EDGES)))
    rsel = jnp.kron(jnp.eye(BOND_DIM, dtype=jnp.float32),
                    jnp.ones((1, ATOM_DIM), jnp.float32))
    ssel = jnp.kron(jnp.ones((1, BOND_DIM), jnp.float32),
                    jnp.eye(ATOM_DIM, dtype=jnp.float32))
    w3 = weight.reshape(BOND_DIM, ATOM_DIM, ATOM_DIM)
    w4 = jnp.transpose(w3, (0, 2, 1)).reshape(BOND_DIM * ATOM_DIM, ATOM_DIM)
    bias_t = bias.reshape(ATOM_DIM, ATOM_DIM).T

    xg = _sc_gather(atom_features, src_pad)
    msg = _tc_matmul(xg.reshape(E_PAD // 4, 128), bond_t, bond_tail_t,
                     rsel, ssel, w4, bias_t)
    msg = msg.reshape(E_PAD, ATOM_DIM)
    zeros_init = jnp.zeros((N_PAD, ATOM_DIM), jnp.float32)
    partials = _sc_scatter(msg, dst_pad, zeros_init)
    out = _tc_combine(partials.reshape(2, N_PAD, ATOM_DIM))
    return out[:N_NODES]


# speedup vs baseline: 1.8485x; 1.1340x over previous
import functools

import jax
import jax.numpy as jnp
from jax import lax
from jax.experimental import pallas as pl
from jax.experimental.pallas import tpu as pltpu
from jax.experimental.pallas import tpu_sc as plsc

ATOM_DIM = 32
BOND_DIM = 16
N_NODES = 10000
N_EDGES = 100000

NW = 32
CHUNK = 128
E_PER_W = 3200
E_PAD = NW * E_PER_W
N_CHUNKS = E_PER_W // CHUNK
N_PAD = 10240
ROWS_PER_TILE = N_PAD // 16


def _permute_idx(idx_lin, idx_p):
    q = E_PER_W // 4

    def body(m, carry):
        k = m * 16 + lax.iota(jnp.int32, 16)
        off = (k >> 2) + (k & 3) * q
        v = plsc.load_gather(idx_lin, [off])
        idx_p[m // 8, pl.ds((m % 8) * 16, 16)] = v
        return carry

    lax.fori_loop(0, E_PER_W // 16, body, 0)


def _gather_body(table_hbm, idx_hbm, out_hbm, idx_lin, idx_p, rows_v, sem):
    cid = lax.axis_index("c")
    sid = lax.axis_index("s")
    wid = sid * 2 + cid
    base = wid * E_PER_W
    pltpu.sync_copy(idx_hbm.at[pl.ds(base, E_PER_W)], idx_lin)
    _permute_idx(idx_lin, idx_p)
    copies = [
        pltpu.async_copy(table_hbm.at[idx_p.at[j]],
                         rows_v.at[pl.ds(j * CHUNK, CHUNK)], sem)
        for j in range(N_CHUNKS)
    ]
    for c in copies:
        c.wait()
    pltpu.sync_copy(rows_v, out_hbm.at[pl.ds(base, E_PER_W)])


def _sc_gather(atom_features, src_idx):
    k = functools.partial(
        pl.kernel,
        out_type=jax.ShapeDtypeStruct((E_PAD, ATOM_DIM), jnp.float32),
        mesh=plsc.VectorSubcoreMesh(core_axis_name="c", subcore_axis_name="s"),
        scratch_types=[
            pltpu.VMEM((E_PER_W,), jnp.int32),
            pltpu.VMEM((N_CHUNKS, CHUNK), jnp.int32),
            pltpu.VMEM((E_PER_W, ATOM_DIM), jnp.float32),
            pltpu.SemaphoreType.DMA,
        ],
        compiler_params=pltpu.CompilerParams(use_tc_tiling_on_sc=False, needs_layout_passes=False),
    )(_gather_body)
    return k(atom_features, src_idx)


def _scatter_body(msg_hbm, idx_hbm, zeros_hbm, out_hbm,
                  msg_v, idx_lin, idx_p, acc_shared):
    cid = lax.axis_index("c")
    sid = lax.axis_index("s")
    wid = sid * 2 + cid
    base = wid * E_PER_W
    row0 = sid * ROWS_PER_TILE
    pltpu.sync_copy(idx_hbm.at[pl.ds(base, E_PER_W)], idx_lin)
    pltpu.sync_copy(msg_hbm.at[pl.ds(base, E_PER_W)], msg_v)
    _permute_idx(idx_lin, idx_p)
    pltpu.sync_copy(zeros_hbm.at[pl.ds(row0, ROWS_PER_TILE)],
                    acc_shared.at[pl.ds(row0, ROWS_PER_TILE)])
    plsc.subcore_barrier()

    def step(j, carry):
        off = j * CHUNK
        pltpu.sync_copy(msg_v.at[pl.ds(off, CHUNK)],
                        acc_shared.at[idx_p.at[j]], add=True)
        return carry

    lax.fori_loop(0, N_CHUNKS, step, 0)
    plsc.subcore_barrier()
    pltpu.sync_copy(acc_shared.at[pl.ds(row0, ROWS_PER_TILE)],
                    out_hbm.at[pl.ds(cid * N_PAD + row0, ROWS_PER_TILE)])


def _sc_scatter(msg, dst_idx, zeros_init):
    k = functools.partial(
        pl.kernel,
        out_type=jax.ShapeDtypeStruct((2 * N_PAD, ATOM_DIM), jnp.float32),
        mesh=plsc.VectorSubcoreMesh(core_axis_name="c", subcore_axis_name="s"),
        scratch_types=[
            pltpu.VMEM((E_PER_W, ATOM_DIM), jnp.float32),
            pltpu.VMEM((E_PER_W,), jnp.int32),
            pltpu.VMEM((N_CHUNKS, CHUNK), jnp.int32),
            pltpu.VMEM_SHARED((N_PAD, ATOM_DIM), jnp.float32),
        ],
        compiler_params=pltpu.CompilerParams(use_tc_tiling_on_sc=False, needs_layout_passes=False),
    )(_scatter_body)
    return k(msg, dst_idx, zeros_init)


def _matmul_body(x_ref, bm_ref, btl_ref, r_ref, s_ref, w4_ref, bt_ref,
                 out_ref):
    z = x_ref[...]
    x = jnp.concatenate([z[:, 32 * c:32 * c + 32] for c in range(4)],
                        axis=0)
    i = pl.program_id(0)
    tile = z.shape[0] * 4
    nfull = N_EDGES // tile
    b = jnp.where(i < nfull, bm_ref[...], btl_ref[...])
    ob = lax.dot_general(b, r_ref[...], (((0,), (0,)), ((), ())),
                         preferred_element_type=jnp.float32)
    o = ob * pltpu.repeat(x, BOND_DIM, axis=1)
    msg = (jnp.dot(o, w4_ref[...], preferred_element_type=jnp.float32)
           + jnp.dot(x, bt_ref[...], preferred_element_type=jnp.float32))
    q = msg.shape[0] // 4
    out_ref[...] = jnp.concatenate(
        [msg[q * c:q * c + q, :] for c in range(4)], axis=1)


def _tc_matmul(xg_packed, bond_t, bond_tail_t, rsel, ssel, w4, bias_t):
    tile = E_PER_W
    grid = (E_PAD // tile,)
    nfull = N_EDGES // tile
    return pl.pallas_call(
        _matmul_body,
        grid=grid,
        in_specs=[
            pl.BlockSpec((tile // 4, 128), lambda i: (i, 0)),
            pl.BlockSpec((BOND_DIM, tile),
                         lambda i: (0, jnp.minimum(i, nfull - 1))),
            pl.BlockSpec((BOND_DIM, tile), lambda i: (0, 0)),
            pl.BlockSpec((BOND_DIM, 512), lambda i: (0, 0)),
            pl.BlockSpec((ATOM_DIM, 512), lambda i: (0, 0)),
            pl.BlockSpec((512, ATOM_DIM), lambda i: (0, 0)),
            pl.BlockSpec((ATOM_DIM, ATOM_DIM), lambda i: (0, 0)),
        ],
        out_specs=pl.BlockSpec((tile // 4, 128), lambda i: (i, 0)),
        out_shape=jax.ShapeDtypeStruct((E_PAD // 4, 128), jnp.float32),
    )(xg_packed, bond_t, bond_tail_t, rsel, ssel, w4, bias_t)


def _combine_body(p_ref, out_ref):
    out_ref[...] = p_ref[0] + p_ref[1]


def _tc_combine(partials):
    return pl.pallas_call(
        _combine_body,
        out_shape=jax.ShapeDtypeStruct((N_PAD, ATOM_DIM), jnp.float32),
    )(partials)


def kernel(atom_features, bond_features, pair_indices, kernel, bias):
    weight = kernel
    src = pair_indices[:, 1].astype(jnp.int32)
    dst = pair_indices[:, 0].astype(jnp.int32)
    src_pad = jnp.concatenate(
        [src, jnp.zeros((E_PAD - N_EDGES,), jnp.int32)])
    dst_pad = jnp.concatenate(
        [dst, jnp.full((E_PAD - N_EDGES,), N_NODES, jnp.int32)])
    tile = E_PER_W
    nfull = N_EDGES // tile
    bond_t = bond_features.T
    bond_tail_t = jnp.pad(bond_t[:, nfull * tile:],
                          ((0, 0), (0, (nfull + 1) * tile - N_EDGES)))
    rsel = jnp.kron(jnp.eye(BOND_DIM, dtype=jnp.float32),
                    jnp.ones((1, ATOM_DIM), jnp.float32))
    ssel = jnp.kron(jnp.ones((1, BOND_DIM), jnp.float32),
                    jnp.eye(ATOM_DIM, dtype=jnp.float32))
    w3 = weight.reshape(BOND_DIM, ATOM_DIM, ATOM_DIM)
    w4 = jnp.transpose(w3, (0, 2, 1)).reshape(BOND_DIM * ATOM_DIM, ATOM_DIM)
    bias_t = bias.reshape(ATOM_DIM, ATOM_DIM).T

    xg = _sc_gather(atom_features, src_pad)
    msg = _tc_matmul(xg.reshape(E_PAD // 4, 128), bond_t, bond_tail_t,
                     rsel, ssel, w4, bias_t)
    msg = msg.reshape(E_PAD, ATOM_DIM)
    zeros_init = jnp.zeros((N_PAD, ATOM_DIM), jnp.float32)
    partials = _sc_scatter(msg, dst_pad, zeros_init)
    out = _tc_combine(partials.reshape(2, N_PAD, ATOM_DIM))
    return out[:N_NODES]


# speedup vs baseline: 1.8790x; 1.0165x over previous
import functools

import jax
import jax.numpy as jnp
from jax import lax
from jax.experimental import pallas as pl
from jax.experimental.pallas import tpu as pltpu
from jax.experimental.pallas import tpu_sc as plsc

ATOM_DIM = 32
BOND_DIM = 16
N_NODES = 10000
N_EDGES = 100000

NW = 32
CHUNK = 128
E_PER_W = 3200
E_PAD = NW * E_PER_W
N_CHUNKS = E_PER_W // CHUNK
N_PAD = 10240
ROWS_PER_TILE = N_PAD // 16


def _permute_idx(idx_lin, idx_p):
    q = E_PER_W // 4

    def body(m, carry):
        k = m * 16 + lax.iota(jnp.int32, 16)
        off = (k >> 2) + (k & 3) * q
        v = plsc.load_gather(idx_lin, [off])
        idx_p[m // 8, pl.ds((m % 8) * 16, 16)] = v
        return carry

    lax.fori_loop(0, E_PER_W // 16, body, 0)


def _gather_body(table_hbm, idx_hbm, out_hbm, idx_lin, idx_p, rows_v,
                 sems, wsem):
    cid = lax.axis_index("c")
    sid = lax.axis_index("s")
    wid = sid * 2 + cid
    base = wid * E_PER_W
    pltpu.sync_copy(idx_hbm.at[pl.ds(base, E_PER_W)], idx_lin)
    _permute_idx(idx_lin, idx_p)
    ngrp = 5
    per = N_CHUNKS // ngrp
    copies = [
        pltpu.async_copy(table_hbm.at[idx_p.at[j]],
                         rows_v.at[pl.ds(j * CHUNK, CHUNK)], sems[j // per])
        for j in range(N_CHUNKS)
    ]
    wb = []
    for g in range(ngrp):
        for c in copies[g * per:(g + 1) * per]:
            c.wait()
        off = g * per * CHUNK
        wb.append(pltpu.async_copy(
            rows_v.at[pl.ds(off, per * CHUNK)],
            out_hbm.at[pl.ds(base + off, per * CHUNK)], wsem))
    for c in wb:
        c.wait()


def _sc_gather(atom_features, src_idx):
    k = functools.partial(
        pl.kernel,
        out_type=jax.ShapeDtypeStruct((E_PAD, ATOM_DIM), jnp.float32),
        mesh=plsc.VectorSubcoreMesh(core_axis_name="c", subcore_axis_name="s"),
        scratch_types=[
            pltpu.VMEM((E_PER_W,), jnp.int32),
            pltpu.VMEM((N_CHUNKS, CHUNK), jnp.int32),
            pltpu.VMEM((E_PER_W, ATOM_DIM), jnp.float32),
            [pltpu.SemaphoreType.DMA] * 5,
            pltpu.SemaphoreType.DMA,
        ],
        compiler_params=pltpu.CompilerParams(use_tc_tiling_on_sc=False, needs_layout_passes=False),
    )(_gather_body)
    return k(atom_features, src_idx)


def _scatter_body(msg_hbm, idx_hbm, zeros_hbm, out_hbm,
                  msg_v, idx_lin, idx_p, acc_shared):
    cid = lax.axis_index("c")
    sid = lax.axis_index("s")
    wid = sid * 2 + cid
    base = wid * E_PER_W
    row0 = sid * ROWS_PER_TILE
    pltpu.sync_copy(idx_hbm.at[pl.ds(base, E_PER_W)], idx_lin)
    pltpu.sync_copy(msg_hbm.at[pl.ds(base, E_PER_W)], msg_v)
    _permute_idx(idx_lin, idx_p)
    pltpu.sync_copy(zeros_hbm.at[pl.ds(row0, ROWS_PER_TILE)],
                    acc_shared.at[pl.ds(row0, ROWS_PER_TILE)])
    plsc.subcore_barrier()

    def step(j, carry):
        off = j * CHUNK
        pltpu.sync_copy(msg_v.at[pl.ds(off, CHUNK)],
                        acc_shared.at[idx_p.at[j]], add=True)
        return carry

    lax.fori_loop(0, N_CHUNKS, step, 0)
    plsc.subcore_barrier()
    pltpu.sync_copy(acc_shared.at[pl.ds(row0, ROWS_PER_TILE)],
                    out_hbm.at[pl.ds(cid * N_PAD + row0, ROWS_PER_TILE)])


def _sc_scatter(msg, dst_idx, zeros_init):
    k = functools.partial(
        pl.kernel,
        out_type=jax.ShapeDtypeStruct((2 * N_PAD, ATOM_DIM), jnp.float32),
        mesh=plsc.VectorSubcoreMesh(core_axis_name="c", subcore_axis_name="s"),
        scratch_types=[
            pltpu.VMEM((E_PER_W, ATOM_DIM), jnp.float32),
            pltpu.VMEM((E_PER_W,), jnp.int32),
            pltpu.VMEM((N_CHUNKS, CHUNK), jnp.int32),
            pltpu.VMEM_SHARED((N_PAD, ATOM_DIM), jnp.float32),
        ],
        compiler_params=pltpu.CompilerParams(use_tc_tiling_on_sc=False, needs_layout_passes=False),
    )(_scatter_body)
    return k(msg, dst_idx, zeros_init)


def _matmul_body(x_ref, bm_ref, btl_ref, r_ref, s_ref, w4_ref, bt_ref,
                 out_ref):
    z = x_ref[...]
    x = jnp.concatenate([z[:, 32 * c:32 * c + 32] for c in range(4)],
                        axis=0)
    i = pl.program_id(0)
    tile = z.shape[0] * 4
    nfull = N_EDGES // tile
    b = jnp.where(i < nfull, bm_ref[...], btl_ref[...])
    ob = lax.dot_general(b, r_ref[...], (((0,), (0,)), ((), ())),
                         preferred_element_type=jnp.float32)
    o = ob * pltpu.repeat(x, BOND_DIM, axis=1)
    msg = (jnp.dot(o, w4_ref[...], preferred_element_type=jnp.float32)
           + jnp.dot(x, bt_ref[...], preferred_element_type=jnp.float32))
    q = msg.shape[0] // 4
    out_ref[...] = jnp.concatenate(
        [msg[q * c:q * c + q, :] for c in range(4)], axis=1)


def _tc_matmul(xg_packed, bond_t, bond_tail_t, rsel, ssel, w4, bias_t):
    tile = E_PER_W
    grid = (E_PAD // tile,)
    nfull = N_EDGES // tile
    return pl.pallas_call(
        _matmul_body,
        grid=grid,
        in_specs=[
            pl.BlockSpec((tile // 4, 128), lambda i: (i, 0)),
            pl.BlockSpec((BOND_DIM, tile),
                         lambda i: (0, jnp.minimum(i, nfull - 1))),
            pl.BlockSpec((BOND_DIM, tile), lambda i: (0, 0)),
            pl.BlockSpec((BOND_DIM, 512), lambda i: (0, 0)),
            pl.BlockSpec((ATOM_DIM, 512), lambda i: (0, 0)),
            pl.BlockSpec((512, ATOM_DIM), lambda i: (0, 0)),
            pl.BlockSpec((ATOM_DIM, ATOM_DIM), lambda i: (0, 0)),
        ],
        out_specs=pl.BlockSpec((tile // 4, 128), lambda i: (i, 0)),
        out_shape=jax.ShapeDtypeStruct((E_PAD // 4, 128), jnp.float32),
    )(xg_packed, bond_t, bond_tail_t, rsel, ssel, w4, bias_t)


def _combine_body(p_ref, out_ref):
    out_ref[...] = p_ref[0] + p_ref[1]


def _tc_combine(partials):
    return pl.pallas_call(
        _combine_body,
        out_shape=jax.ShapeDtypeStruct((N_PAD, ATOM_DIM), jnp.float32),
    )(partials)


def kernel(atom_features, bond_features, pair_indices, kernel, bias):
    weight = kernel
    src = pair_indices[:, 1].astype(jnp.int32)
    dst = pair_indices[:, 0].astype(jnp.int32)
    src_pad = jnp.concatenate(
        [src, jnp.zeros((E_PAD - N_EDGES,), jnp.int32)])
    dst_pad = jnp.concatenate(
        [dst, jnp.full((E_PAD - N_EDGES,), N_NODES, jnp.int32)])
    tile = E_PER_W
    nfull = N_EDGES // tile
    bond_t = bond_features.T
    bond_tail_t = jnp.pad(bond_t[:, nfull * tile:],
                          ((0, 0), (0, (nfull + 1) * tile - N_EDGES)))
    rsel = jnp.kron(jnp.eye(BOND_DIM, dtype=jnp.float32),
                    jnp.ones((1, ATOM_DIM), jnp.float32))
    ssel = jnp.kron(jnp.ones((1, BOND_DIM), jnp.float32),
                    jnp.eye(ATOM_DIM, dtype=jnp.float32))
    w3 = weight.reshape(BOND_DIM, ATOM_DIM, ATOM_DIM)
    w4 = jnp.transpose(w3, (0, 2, 1)).reshape(BOND_DIM * ATOM_DIM, ATOM_DIM)
    bias_t = bias.reshape(ATOM_DIM, ATOM_DIM).T

    xg = _sc_gather(atom_features, src_pad)
    msg = _tc_matmul(xg.reshape(E_PAD // 4, 128), bond_t, bond_tail_t,
                     rsel, ssel, w4, bias_t)
    msg = msg.reshape(E_PAD, ATOM_DIM)
    zeros_init = jnp.zeros((N_PAD, ATOM_DIM), jnp.float32)
    partials = _sc_scatter(msg, dst_pad, zeros_init)
    out = _tc_combine(partials.reshape(2, N_PAD, ATOM_DIM))
    return out[:N_NODES]


# speedup vs baseline: 2.0099x; 1.0697x over previous
import functools

import jax
import jax.numpy as jnp
from jax import lax
from jax.experimental import pallas as pl
from jax.experimental.pallas import tpu as pltpu
from jax.experimental.pallas import tpu_sc as plsc

ATOM_DIM = 32
BOND_DIM = 16
N_NODES = 10000
N_EDGES = 100000

NW = 32
CHUNK = 128
E_PER_W = 3200
E_PAD = NW * E_PER_W
N_CHUNKS = E_PER_W // CHUNK
N_PAD = 10240
ROWS_PER_TILE = N_PAD // 16


def _permute_idx(idx_lin, idx_p, permute_values=False):
    q = E_PER_W // 4

    def body(m, carry):
        k = m * 16 + lax.iota(jnp.int32, 16)
        off = (k >> 2) + (k & 3) * q
        v = plsc.load_gather(idx_lin, [off])
        if permute_values:
            loc = v & 2047
            v = (v & ~jnp.int32(2047)) | ((loc & 511) << 2) | (loc >> 9)
        idx_p[m // 8, pl.ds((m % 8) * 16, 16)] = v
        return carry

    lax.fori_loop(0, E_PER_W // 16, body, 0)


def _gather_body(table_hbm, idx_hbm, out_hbm, idx_lin, idx_p, rows_v,
                 sems, wsem):
    cid = lax.axis_index("c")
    sid = lax.axis_index("s")
    wid = sid * 2 + cid
    base = wid * E_PER_W
    pltpu.sync_copy(idx_hbm.at[pl.ds(base, E_PER_W)], idx_lin)
    _permute_idx(idx_lin, idx_p)
    ngrp = 5
    per = N_CHUNKS // ngrp
    copies = [
        pltpu.async_copy(table_hbm.at[idx_p.at[j]],
                         rows_v.at[pl.ds(j * CHUNK, CHUNK)], sems[j // per])
        for j in range(N_CHUNKS)
    ]
    wb = []
    for g in range(ngrp):
        for c in copies[g * per:(g + 1) * per]:
            c.wait()
        off = g * per * CHUNK
        wb.append(pltpu.async_copy(
            rows_v.at[pl.ds(off, per * CHUNK)],
            out_hbm.at[pl.ds(base + off, per * CHUNK)], wsem))
    for c in wb:
        c.wait()


def _sc_gather(atom_features, src_idx):
    k = functools.partial(
        pl.kernel,
        out_type=jax.ShapeDtypeStruct((E_PAD, ATOM_DIM), jnp.float32),
        mesh=plsc.VectorSubcoreMesh(core_axis_name="c", subcore_axis_name="s"),
        scratch_types=[
            pltpu.VMEM((E_PER_W,), jnp.int32),
            pltpu.VMEM((N_CHUNKS, CHUNK), jnp.int32),
            pltpu.VMEM((E_PER_W, ATOM_DIM), jnp.float32),
            [pltpu.SemaphoreType.DMA] * 5,
            pltpu.SemaphoreType.DMA,
        ],
        compiler_params=pltpu.CompilerParams(use_tc_tiling_on_sc=False, needs_layout_passes=False),
    )(_gather_body)
    return k(atom_features, src_idx)


def _scatter_body(msg_hbm, idx_hbm, zeros_hbm, out_hbm,
                  msg_v, idx_lin, idx_p, acc_shared):
    cid = lax.axis_index("c")
    sid = lax.axis_index("s")
    wid = sid * 2 + cid
    base = wid * E_PER_W
    row0 = sid * ROWS_PER_TILE
    pltpu.sync_copy(idx_hbm.at[pl.ds(base, E_PER_W)], idx_lin)
    pltpu.sync_copy(msg_hbm.at[pl.ds(base, E_PER_W)], msg_v)
    _permute_idx(idx_lin, idx_p, permute_values=True)
    pltpu.sync_copy(zeros_hbm.at[pl.ds(row0, ROWS_PER_TILE)],
                    acc_shared.at[pl.ds(row0, ROWS_PER_TILE)])
    plsc.subcore_barrier()

    def step(j, carry):
        off = j * CHUNK
        pltpu.sync_copy(msg_v.at[pl.ds(off, CHUNK)],
                        acc_shared.at[idx_p.at[j]], add=True)
        return carry

    lax.fori_loop(0, N_CHUNKS, step, 0)
    plsc.subcore_barrier()
    pltpu.sync_copy(acc_shared.at[pl.ds(row0, ROWS_PER_TILE)],
                    out_hbm.at[pl.ds(cid * N_PAD + row0, ROWS_PER_TILE)])


def _sc_scatter(msg, dst_idx, zeros_init):
    k = functools.partial(
        pl.kernel,
        out_type=jax.ShapeDtypeStruct((2 * N_PAD, ATOM_DIM), jnp.float32),
        mesh=plsc.VectorSubcoreMesh(core_axis_name="c", subcore_axis_name="s"),
        scratch_types=[
            pltpu.VMEM((E_PER_W, ATOM_DIM), jnp.float32),
            pltpu.VMEM((E_PER_W,), jnp.int32),
            pltpu.VMEM((N_CHUNKS, CHUNK), jnp.int32),
            pltpu.VMEM_SHARED((N_PAD, ATOM_DIM), jnp.float32),
        ],
        compiler_params=pltpu.CompilerParams(use_tc_tiling_on_sc=False, needs_layout_passes=False),
    )(_scatter_body)
    return k(msg, dst_idx, zeros_init)


def _matmul_body(x_ref, bm_ref, btl_ref, r_ref, s_ref, w4_ref, bt_ref,
                 out_ref):
    z = x_ref[...]
    x = jnp.concatenate([z[:, 32 * c:32 * c + 32] for c in range(4)],
                        axis=0)
    i = pl.program_id(0)
    tile = z.shape[0] * 4
    nfull = N_EDGES // tile
    b = jnp.where(i < nfull, bm_ref[...], btl_ref[...])
    ob = lax.dot_general(b, r_ref[...], (((0,), (0,)), ((), ())),
                         preferred_element_type=jnp.float32)
    o = ob * pltpu.repeat(x, BOND_DIM, axis=1)
    msg = (jnp.dot(o, w4_ref[...], preferred_element_type=jnp.float32)
           + jnp.dot(x, bt_ref[...], preferred_element_type=jnp.float32))
    q = msg.shape[0] // 4
    out_ref[...] = jnp.concatenate(
        [msg[q * c:q * c + q, :] for c in range(4)], axis=1)


def _tc_matmul(xg_packed, bond_t, bond_tail_t, rsel, ssel, w4, bias_t):
    tile = E_PER_W
    grid = (E_PAD // tile,)
    nfull = N_EDGES // tile
    return pl.pallas_call(
        _matmul_body,
        grid=grid,
        in_specs=[
            pl.BlockSpec((tile // 4, 128), lambda i: (i, 0)),
            pl.BlockSpec((BOND_DIM, tile),
                         lambda i: (0, jnp.minimum(i, nfull - 1))),
            pl.BlockSpec((BOND_DIM, tile), lambda i: (0, 0)),
            pl.BlockSpec((BOND_DIM, 512), lambda i: (0, 0)),
            pl.BlockSpec((ATOM_DIM, 512), lambda i: (0, 0)),
            pl.BlockSpec((512, ATOM_DIM), lambda i: (0, 0)),
            pl.BlockSpec((ATOM_DIM, ATOM_DIM), lambda i: (0, 0)),
        ],
        out_specs=pl.BlockSpec((tile // 4, 128), lambda i: (i, 0)),
        out_shape=jax.ShapeDtypeStruct((E_PAD // 4, 128), jnp.float32),
    )(xg_packed, bond_t, bond_tail_t, rsel, ssel, w4, bias_t)


def _combine_body(p_ref, out_ref):
    s = p_ref[0] + p_ref[1]
    out_ref[...] = jnp.concatenate(
        [s[:, 32 * c:32 * c + 32] for c in range(4)], axis=0)


def _tc_combine(partials_packed):
    return pl.pallas_call(
        _combine_body,
        grid=(N_PAD // 2048,),
        in_specs=[pl.BlockSpec((2, 512, 128), lambda i: (0, i, 0))],
        out_specs=pl.BlockSpec((2048, ATOM_DIM), lambda i: (i, 0)),
        out_shape=jax.ShapeDtypeStruct((N_NODES, ATOM_DIM), jnp.float32),
    )(partials_packed)


def kernel(atom_features, bond_features, pair_indices, kernel, bias):
    weight = kernel
    src = pair_indices[:, 1].astype(jnp.int32)
    dst = pair_indices[:, 0].astype(jnp.int32)
    src_pad = jnp.concatenate(
        [src, jnp.zeros((E_PAD - N_EDGES,), jnp.int32)])
    dst_pad = jnp.concatenate(
        [dst, jnp.full((E_PAD - N_EDGES,), N_NODES, jnp.int32)])
    tile = E_PER_W
    nfull = N_EDGES // tile
    bond_t = bond_features.T
    bond_tail_t = jnp.pad(bond_t[:, nfull * tile:],
                          ((0, 0), (0, (nfull + 1) * tile - N_EDGES)))
    rsel = jnp.kron(jnp.eye(BOND_DIM, dtype=jnp.float32),
                    jnp.ones((1, ATOM_DIM), jnp.float32))
    ssel = jnp.kron(jnp.ones((1, BOND_DIM), jnp.float32),
                    jnp.eye(ATOM_DIM, dtype=jnp.float32))
    w3 = weight.reshape(BOND_DIM, ATOM_DIM, ATOM_DIM)
    w4 = jnp.transpose(w3, (0, 2, 1)).reshape(BOND_DIM * ATOM_DIM, ATOM_DIM)
    bias_t = bias.reshape(ATOM_DIM, ATOM_DIM).T

    xg = _sc_gather(atom_features, src_pad)
    msg = _tc_matmul(xg.reshape(E_PAD // 4, 128), bond_t, bond_tail_t,
                     rsel, ssel, w4, bias_t)
    msg = msg.reshape(E_PAD, ATOM_DIM)
    zeros_init = jnp.zeros((N_PAD, ATOM_DIM), jnp.float32)
    partials = _sc_scatter(msg, dst_pad, zeros_init)
    return _tc_combine(partials.reshape(2, N_PAD // 4, 128))
